# Initial kernel scaffold; baseline (speedup 1.0000x reference)
import jax
import jax.numpy as jnp
from jax.experimental import pallas as pl


def kernel(data, indices):
    raise NotImplementedError("write your pallas kernel here")


# speedup vs baseline: 1.0935x; 1.0935x over previous
import functools

import jax
import jax.numpy as jnp
from jax import lax
from jax.experimental import pallas as pl
from jax.experimental.pallas import tpu as pltpu
from jax.experimental.pallas import tpu_sc as plsc

_NUM_CORES = 2
_NUM_SUBCORES = 16
_NUM_WORKERS = _NUM_CORES * _NUM_SUBCORES
_CHUNK = 1024


@functools.partial(jax.jit, static_argnames=())
def kernel(data, indices):
    batch, hist = indices.shape
    vocab, dim = data.shape
    n = batch * hist
    assert n % _NUM_WORKERS == 0
    per_w = n // _NUM_WORKERS
    assert per_w % _CHUNK == 0
    n_chunks = per_w // _CHUNK

    flat_idx = indices.reshape(n)

    mesh = plsc.VectorSubcoreMesh(core_axis_name="c", subcore_axis_name="s")

    @functools.partial(
        pl.kernel,
        out_type=jax.ShapeDtypeStruct((n, dim), jnp.float32),
        mesh=mesh,
        scratch_types=[
            pltpu.VMEM((_CHUNK,), jnp.int32),
            pltpu.VMEM((_CHUNK, dim), jnp.float32),
            pltpu.SemaphoreType.DMA,
        ],
        compiler_params=pltpu.CompilerParams(use_tc_tiling_on_sc=False),
    )
    def gather_kernel(table_hbm, idx_hbm, out_hbm, idx_v, rows_v, sem):
        wid = lax.axis_index("s") * _NUM_CORES + lax.axis_index("c")
        base = wid * per_w

        def body(g, carry):
            off = pl.multiple_of(base + g * _CHUNK, _CHUNK)
            pltpu.sync_copy(idx_hbm.at[pl.ds(off, _CHUNK)], idx_v)
            pltpu.async_copy(table_hbm.at[idx_v], rows_v, sem).wait()
            pltpu.sync_copy(rows_v, out_hbm.at[pl.ds(off, _CHUNK)])
            return carry

        lax.fori_loop(0, n_chunks, body, 0)

    out_flat = gather_kernel(data, flat_idx)
    return out_flat.reshape(batch, hist, dim)


# speedup vs baseline: 1.1121x; 1.0171x over previous
import functools

import jax
import jax.numpy as jnp
from jax import lax
from jax.experimental import pallas as pl
from jax.experimental.pallas import tpu as pltpu
from jax.experimental.pallas import tpu_sc as plsc

_NUM_CORES = 2
_NUM_SUBCORES = 16
_NUM_WORKERS = _NUM_CORES * _NUM_SUBCORES
_CHUNK = 1600


def kernel(data, indices):
    batch, hist = indices.shape
    vocab, dim = data.shape
    n = batch * hist
    assert n % _NUM_WORKERS == 0
    per_w = n // _NUM_WORKERS
    assert per_w % _CHUNK == 0
    n_chunks = per_w // _CHUNK

    flat_idx = indices.reshape(n)

    mesh = plsc.VectorSubcoreMesh(core_axis_name="c", subcore_axis_name="s")

    @functools.partial(
        pl.kernel,
        out_type=jax.ShapeDtypeStruct((n, dim), jnp.float32),
        mesh=mesh,
        scratch_types=[
            pltpu.VMEM((per_w,), jnp.int32),
            pltpu.VMEM((_CHUNK, dim), jnp.float32),
            pltpu.VMEM((_CHUNK, dim), jnp.float32),
            pltpu.SemaphoreType.DMA,
            pltpu.SemaphoreType.DMA,
            pltpu.SemaphoreType.DMA,
            pltpu.SemaphoreType.DMA,
        ],
        compiler_params=pltpu.CompilerParams(use_tc_tiling_on_sc=False),
    )
    def gather_kernel(table_hbm, idx_hbm, out_hbm, idx_all, rows0, rows1,
                      gsem0, gsem1, wsem0, wsem1):
        wid = lax.axis_index("s") * _NUM_CORES + lax.axis_index("c")
        base = wid * per_w
        rows = (rows0, rows1)
        gsem = (gsem0, gsem1)
        wsem = (wsem0, wsem1)

        pltpu.sync_copy(idx_hbm.at[pl.ds(pl.multiple_of(base, per_w), per_w)],
                        idx_all)

        gd = [None] * n_chunks
        wd = [None] * n_chunks
        for g in range(n_chunks):
            b = g % 2
            if g >= 2:
                wd[g - 2].wait()
            gd[g] = pltpu.async_copy(
                table_hbm.at[idx_all.at[pl.ds(g * _CHUNK, _CHUNK)]],
                rows[b], gsem[b])
            if g >= 1:
                gd[g - 1].wait()
                off = pl.multiple_of(base + (g - 1) * _CHUNK, _CHUNK)
                wd[g - 1] = pltpu.async_copy(
                    rows[1 - b], out_hbm.at[pl.ds(off, _CHUNK)], wsem[1 - b])
        last = n_chunks - 1
        gd[last].wait()
        off = pl.multiple_of(base + last * _CHUNK, _CHUNK)
        wd[last] = pltpu.async_copy(rows[last % 2],
                                    out_hbm.at[pl.ds(off, _CHUNK)],
                                    wsem[last % 2])
        wd[last - 1].wait()
        wd[last].wait()

    out_flat = gather_kernel(data, flat_idx)
    return out_flat.reshape(batch, hist, dim)


# speedup vs baseline: 1.1125x; 1.0003x over previous
import functools

import jax
import jax.numpy as jnp
from jax import lax
from jax.experimental import pallas as pl
from jax.experimental.pallas import tpu as pltpu
from jax.experimental.pallas import tpu_sc as plsc

_NUM_CORES = 2
_NUM_SUBCORES = 16
_NUM_WORKERS = _NUM_CORES * _NUM_SUBCORES
_CHUNK = 800
_DEPTH = 4


def kernel(data, indices):
    batch, hist = indices.shape
    vocab, dim = data.shape
    n = batch * hist
    assert n % _NUM_WORKERS == 0
    per_w = n // _NUM_WORKERS
    assert per_w % _CHUNK == 0
    n_chunks = per_w // _CHUNK
    assert n_chunks >= _DEPTH

    flat_idx = indices.reshape(n)

    mesh = plsc.VectorSubcoreMesh(core_axis_name="c", subcore_axis_name="s")

    scratch = [pltpu.VMEM((per_w,), jnp.int32)]
    scratch += [pltpu.VMEM((_CHUNK, dim), jnp.float32)] * _DEPTH
    scratch += [pltpu.SemaphoreType.DMA] * (2 * _DEPTH)

    @functools.partial(
        pl.kernel,
        out_type=jax.ShapeDtypeStruct((n, dim), jnp.float32),
        mesh=mesh,
        scratch_types=scratch,
        compiler_params=pltpu.CompilerParams(use_tc_tiling_on_sc=False),
    )
    def gather_kernel(table_hbm, idx_hbm, out_hbm, idx_all, *bufs):
        rows = bufs[:_DEPTH]
        gsem = bufs[_DEPTH:2 * _DEPTH]
        wsem = bufs[2 * _DEPTH:]
        wid = lax.axis_index("s") * _NUM_CORES + lax.axis_index("c")
        base = wid * per_w

        pltpu.sync_copy(idx_hbm.at[pl.ds(pl.multiple_of(base, per_w), per_w)],
                        idx_all)

        gd = [None] * n_chunks
        wd = [None] * n_chunks

        def write(h):
            gd[h].wait()
            off = pl.multiple_of(base + h * _CHUNK, _CHUNK)
            wd[h] = pltpu.async_copy(
                rows[h % _DEPTH], out_hbm.at[pl.ds(off, _CHUNK)],
                wsem[h % _DEPTH])

        for g in range(n_chunks):
            b = g % _DEPTH
            if g >= _DEPTH:
                wd[g - _DEPTH].wait()
            gd[g] = pltpu.async_copy(
                table_hbm.at[idx_all.at[pl.ds(g * _CHUNK, _CHUNK)]],
                rows[b], gsem[b])
            if g >= _DEPTH - 1:
                write(g - (_DEPTH - 1))
        for h in range(n_chunks - (_DEPTH - 1), n_chunks):
            write(h)
        for h in range(n_chunks - _DEPTH, n_chunks):
            wd[h].wait()

    out_flat = gather_kernel(data, flat_idx)
    return out_flat.reshape(batch, hist, dim)


# speedup vs baseline: 1.7507x; 1.5737x over previous
import functools

import jax
import jax.numpy as jnp
from jax import lax
from jax.experimental import pallas as pl
from jax.experimental.pallas import tpu as pltpu
from jax.experimental.pallas import tpu_sc as plsc

_NUM_CORES = 2
_NUM_SUBCORES = 16
_NUM_WORKERS = _NUM_CORES * _NUM_SUBCORES
_CB = 16


def kernel(data, indices):
    batch, hist = indices.shape
    vocab, dim = data.shape
    assert batch % _NUM_WORKERS == 0
    b_per_w = batch // _NUM_WORKERS
    assert b_per_w % _CB == 0
    n_chunks = b_per_w // _CB
    hist_pad = 64
    per_w = b_per_w * hist_pad

    flat_idx = jnp.pad(indices, ((0, 0), (0, hist_pad - hist))).reshape(
        batch * hist_pad)

    mesh = plsc.VectorSubcoreMesh(core_axis_name="c", subcore_axis_name="s")

    @functools.partial(
        pl.kernel,
        out_type=jax.ShapeDtypeStruct((batch, hist, dim), jnp.float32),
        mesh=mesh,
        scratch_types=[
            pltpu.VMEM((per_w,), jnp.int32),
            pltpu.VMEM((_CB, hist, dim), jnp.float32),
        ]
        + [pltpu.SemaphoreType.DMA] * (_CB + 1),
        compiler_params=pltpu.CompilerParams(use_tc_tiling_on_sc=False),
    )
    def gather_kernel(table_hbm, idx_hbm, out_hbm, idx_all, rows3, *sems):
        gsems = sems[:_CB]
        wsem = sems[_CB]
        wid = lax.axis_index("s") * _NUM_CORES + lax.axis_index("c")
        base = wid * per_w
        bbase = wid * b_per_w

        pltpu.sync_copy(idx_hbm.at[pl.ds(pl.multiple_of(base, per_w), per_w)],
                        idx_all)

        def body(g, carry):
            gds = []
            for i in range(_CB):
                gds.append(pltpu.async_copy(
                    table_hbm.at[idx_all.at[pl.ds(
                        g * (_CB * hist_pad) + i * hist_pad, hist)]],
                    rows3.at[i], gsems[i]))
            for gd in gds:
                gd.wait()
            pltpu.async_copy(
                rows3, out_hbm.at[pl.ds(bbase + g * _CB, _CB)], wsem).wait()
            return carry

        lax.fori_loop(0, n_chunks, body, 0)

    return gather_kernel(data, flat_idx)


# speedup vs baseline: 1.7719x; 1.0121x over previous
import functools

import jax
import jax.numpy as jnp
from jax import lax
from jax.experimental import pallas as pl
from jax.experimental.pallas import tpu as pltpu
from jax.experimental.pallas import tpu_sc as plsc

_NUM_CORES = 2
_NUM_SUBCORES = 16
_NUM_WORKERS = _NUM_CORES * _NUM_SUBCORES
_CB = 16


def kernel(data, indices):
    batch, hist = indices.shape
    vocab, dim = data.shape
    assert batch % _NUM_WORKERS == 0
    b_per_w = batch // _NUM_WORKERS
    assert b_per_w % (2 * _CB) == 0
    n_chunks = b_per_w // _CB
    hist_pad = 64
    per_w = b_per_w * hist_pad

    flat_idx = jnp.pad(indices, ((0, 0), (0, hist_pad - hist))).reshape(
        batch * hist_pad)

    mesh = plsc.VectorSubcoreMesh(core_axis_name="c", subcore_axis_name="s")

    @functools.partial(
        pl.kernel,
        out_type=jax.ShapeDtypeStruct((batch, hist, dim), jnp.float32),
        mesh=mesh,
        scratch_types=[
            pltpu.VMEM((per_w,), jnp.int32),
            pltpu.VMEM((_CB, hist, dim), jnp.float32),
            pltpu.VMEM((_CB, hist, dim), jnp.float32),
            pltpu.SemaphoreType.DMA,
            pltpu.SemaphoreType.DMA,
            pltpu.SemaphoreType.DMA,
            pltpu.SemaphoreType.DMA,
        ],
        compiler_params=pltpu.CompilerParams(use_tc_tiling_on_sc=False),
    )
    def gather_kernel(table_hbm, idx_hbm, out_hbm, idx_all, rows0, rows1,
                      gsem0, gsem1, wsem0, wsem1):
        rows = (rows0, rows1)
        gsems = (gsem0, gsem1)
        wsems = (wsem0, wsem1)
        wid = lax.axis_index("s") * _NUM_CORES + lax.axis_index("c")
        base = wid * per_w
        bbase = wid * b_per_w

        pltpu.sync_copy(idx_hbm.at[pl.ds(pl.multiple_of(base, per_w), per_w)],
                        idx_all)

        def gather_chunk(g, b):
            gds = []
            for i in range(_CB):
                gds.append(pltpu.async_copy(
                    table_hbm.at[idx_all.at[pl.ds(
                        g * (_CB * hist_pad) + i * hist_pad, hist)]],
                    rows[b].at[i], gsems[b]))
            return gds

        def write_chunk(g, b):
            return pltpu.async_copy(
                rows[b], out_hbm.at[pl.ds(bbase + g * _CB, _CB)], wsems[b])

        def body(h, carry):
            g0 = h * 2
            gds0 = gather_chunk(g0, 0)
            for gd in gds0:
                gd.wait()
            wd0 = write_chunk(g0, 0)
            gds1 = gather_chunk(g0 + 1, 1)
            for gd in gds1:
                gd.wait()
            wd0.wait()
            wd1 = write_chunk(g0 + 1, 1)
            wd1.wait()
            return carry

        lax.fori_loop(0, n_chunks // 2, body, 0)

    return gather_kernel(data, flat_idx)
